# baseline (speedup 1.0000x reference)
import functools

import jax
import jax.numpy as jnp
from jax import lax
from jax.experimental import pallas as pl
from jax.experimental.pallas import tpu as pltpu
from jax.experimental.pallas import tpu_sc as plsc

N_VERTS = 6890
N_FACES = 13776
B = 64

NC = 2
NS = 16
NW = NC * NS
L = 16

K = 40
ITERS = 11
ROWD = 6 * (B // 2)
NB = B // L
NG = 2


def _edge_body(table_hbm, idxs_hbm, mask_hbm, out_hbm,
               idx_v, buf_v, mask_v, out_v, sem0, sem1):
    cid = lax.axis_index("c")
    sid = lax.axis_index("s")
    w = sid * NC + cid

    pltpu.sync_copy(idxs_hbm.at[w], idx_v)
    pltpu.sync_copy(mask_hbm, mask_v)
    accs = tuple(jnp.zeros((L,), jnp.float32) for _ in range(NB))

    sems = (sem0, sem1)
    pending = pltpu.async_copy(table_hbm.at[idx_v.at[0]], buf_v.at[0], sem0)
    for it in range(ITERS):
        slot = it % 2
        cur = pending
        if it + 1 < ITERS:
            pending = pltpu.async_copy(
                table_hbm.at[idx_v.at[it + 1]],
                buf_v.at[(it + 1) % 2],
                sems[(it + 1) % 2],
            )
        cur.wait()

        def face_body(k, accs, slot=slot):
            out = list(accs)
            for g in range(NG):
                o = g * L

                def ldrow(r):
                    return [plsc.bitcast(
                        buf_v[slot, r, pl.ds(d * 2 * L + o, L)], jnp.bfloat16)
                        for d in range(6)]

                v1 = ldrow(k)
                v2 = ldrow(K + k)
                v3 = ldrow(2 * K + k)
                e12p = (jnp.abs(v1[0] - v2[0]) + jnp.abs(v1[1] - v2[1])
                        + jnp.abs(v1[2] - v2[2]))
                e13p = (jnp.abs(v1[0] - v3[0]) + jnp.abs(v1[1] - v3[1])
                        + jnp.abs(v1[2] - v3[2]))
                e23p = (jnp.abs(v2[0] - v3[0]) + jnp.abs(v2[1] - v3[1])
                        + jnp.abs(v2[2] - v3[2]))
                e12g = (jnp.abs(v1[3] - v2[3]) + jnp.abs(v1[4] - v2[4])
                        + jnp.abs(v1[5] - v2[5]))
                e13g = (jnp.abs(v1[3] - v3[3]) + jnp.abs(v1[4] - v3[4])
                        + jnp.abs(v1[5] - v3[5]))
                e23g = (jnp.abs(v2[3] - v3[3]) + jnp.abs(v2[4] - v3[4])
                        + jnp.abs(v2[5] - v3[5]))
                t = (jnp.abs(e12p - e12g) + jnp.abs(e13p - e13g)
                     + jnp.abs(e23p - e23g))
                ta, tb = plsc.unpack(t, format=plsc.PackFormat.INTERLEAVED)
                out[g * 2] = out[g * 2] + ta
                out[g * 2 + 1] = out[g * 2 + 1] + tb
            return tuple(out)

        accs = lax.fori_loop(0, K, face_body, accs)

    part = accs[0] * mask_v[pl.ds(0, L)]
    msum = mask_v[pl.ds(0, L)]
    for cc in range(1, NB):
        part = part + accs[cc] * mask_v[pl.ds(cc * L, L)]
        msum = msum + mask_v[pl.ds(cc * L, L)]
    cs = jnp.flip(plsc.cumsum(msum))
    lane = lax.iota(jnp.int32, L)
    total = plsc.cumsum(jnp.where(lane == 0, cs, jnp.float32(0.0)))
    denom = total * jnp.float32(N_FACES)
    out_v[...] = part / denom
    pltpu.sync_copy(out_v, out_hbm.at[w])


@jax.jit
def _edge_loss(table, idxs, maskf):
    mesh = plsc.VectorSubcoreMesh(core_axis_name="c", subcore_axis_name="s")
    run = functools.partial(
        pl.kernel,
        out_type=jax.ShapeDtypeStruct((NW, L), jnp.float32),
        mesh=mesh,
        compiler_params=pltpu.CompilerParams(
            needs_layout_passes=False, use_tc_tiling_on_sc=False),
        scratch_types=[
            pltpu.VMEM((ITERS, 3 * K), jnp.int32),
            pltpu.VMEM((2, 3 * K, ROWD), jnp.float32),
            pltpu.VMEM((B,), jnp.float32),
            pltpu.VMEM((L,), jnp.float32),
            pltpu.SemaphoreType.DMA,
            pltpu.SemaphoreType.DMA,
        ],
    )(_edge_body)
    out = run(table, idxs, maskf)
    return jnp.sum(out)


def _pack(x):
    xt = x.astype(jnp.bfloat16).transpose(1, 2, 0)
    u = lax.bitcast_convert_type(xt, jnp.uint16).astype(jnp.uint32)
    words = u[..., 0::2] | (u[..., 1::2] << 16)
    return lax.bitcast_convert_type(words, jnp.float32).reshape(N_VERTS, -1)


def kernel(pred_verts, gt_verts, flag, faces):
    table = jnp.concatenate([_pack(pred_verts), _pack(gt_verts)], axis=1)
    f = faces.astype(jnp.int32)
    pad = NW * ITERS * K - N_FACES
    fp = jnp.concatenate([f, jnp.zeros((pad, 3), jnp.int32)], axis=0)
    idxs = (fp.reshape(NW, ITERS, K, 3)
            .transpose(0, 1, 3, 2)
            .reshape(NW, ITERS, 3 * K))
    maskf = (flag == 1).astype(jnp.float32)
    maskp = maskf.reshape(NG, L, 2).transpose(0, 2, 1).reshape(B)
    return _edge_loss(table, idxs, maskp)


# speedup vs baseline: 1.2768x; 1.2768x over previous
import functools

import jax
import jax.numpy as jnp
from jax import lax
from jax.experimental import pallas as pl
from jax.experimental.pallas import tpu as pltpu
from jax.experimental.pallas import tpu_sc as plsc

N_VERTS = 6890
N_FACES = 13776
B = 64

NC = 2
NS = 16
NW = NC * NS
L = 16

K = 40
IT0 = 16
IT1 = 6
MAXIT = 16
ROWD = 6 * (B // 2)
NB = B // L
NG = 2


def _chunk_body(buf_v, slot, k, accs):
    out = list(accs)
    for g in range(NG):
        o = g * L

        def ldrow(r):
            return [plsc.bitcast(
                buf_v[slot, r, pl.ds(d * 2 * L + o, L)], jnp.bfloat16)
                for d in range(6)]

        v1 = ldrow(k)
        v2 = ldrow(K + k)
        v3 = ldrow(2 * K + k)
        e12p = (jnp.abs(v1[0] - v2[0]) + jnp.abs(v1[1] - v2[1])
                + jnp.abs(v1[2] - v2[2]))
        e13p = (jnp.abs(v1[0] - v3[0]) + jnp.abs(v1[1] - v3[1])
                + jnp.abs(v1[2] - v3[2]))
        e23p = (jnp.abs(v2[0] - v3[0]) + jnp.abs(v2[1] - v3[1])
                + jnp.abs(v2[2] - v3[2]))
        e12g = (jnp.abs(v1[3] - v2[3]) + jnp.abs(v1[4] - v2[4])
                + jnp.abs(v1[5] - v2[5]))
        e13g = (jnp.abs(v1[3] - v3[3]) + jnp.abs(v1[4] - v3[4])
                + jnp.abs(v1[5] - v3[5]))
        e23g = (jnp.abs(v2[3] - v3[3]) + jnp.abs(v2[4] - v3[4])
                + jnp.abs(v2[5] - v3[5]))
        t = (jnp.abs(e12p - e12g) + jnp.abs(e13p - e13g)
             + jnp.abs(e23p - e23g))
        ta, tb = plsc.unpack(t, format=plsc.PackFormat.INTERLEAVED)
        out[g * 2] = out[g * 2] + ta
        out[g * 2 + 1] = out[g * 2 + 1] + tb
    return tuple(out)


def _edge_body(table_hbm, idxs_hbm, mask_hbm, out_hbm,
               idx_v, buf_v, mask_v, acc_v, out_v, sem0, sem1):
    cid = lax.axis_index("c")
    sid = lax.axis_index("s")
    w = sid * NC + cid

    pltpu.sync_copy(idxs_hbm.at[w], idx_v)
    pltpu.sync_copy(mask_hbm, mask_v)

    sems = (sem0, sem1)

    def run_chunks(iters):
        accs = tuple(jnp.zeros((L,), jnp.float32) for _ in range(NB))
        pending = pltpu.async_copy(table_hbm.at[idx_v.at[0]], buf_v.at[0],
                                   sem0)
        for it in range(iters):
            slot = it % 2
            cur = pending
            if it + 1 < iters:
                pending = pltpu.async_copy(
                    table_hbm.at[idx_v.at[it + 1]],
                    buf_v.at[(it + 1) % 2],
                    sems[(it + 1) % 2],
                )
            cur.wait()

            def face_body(k, accs, slot=slot):
                return _chunk_body(buf_v, slot, k, accs)

            accs = lax.fori_loop(0, K, face_body, accs)
        for cc in range(NB):
            acc_v[cc, :] = accs[cc]

    @pl.when(cid == 0)
    def _():
        run_chunks(IT0)

    @pl.when(cid != 0)
    def _():
        run_chunks(IT1)

    part = acc_v[0, :] * mask_v[pl.ds(0, L)]
    msum = mask_v[pl.ds(0, L)]
    for cc in range(1, NB):
        part = part + acc_v[cc, :] * mask_v[pl.ds(cc * L, L)]
        msum = msum + mask_v[pl.ds(cc * L, L)]
    cs = jnp.flip(plsc.cumsum(msum))
    lane = lax.iota(jnp.int32, L)
    total = plsc.cumsum(jnp.where(lane == 0, cs, jnp.float32(0.0)))
    denom = total * jnp.float32(N_FACES)
    out_v[...] = part / denom
    pltpu.sync_copy(out_v, out_hbm.at[w])


@jax.jit
def _edge_loss(table, idxs, maskf):
    mesh = plsc.VectorSubcoreMesh(core_axis_name="c", subcore_axis_name="s")
    run = functools.partial(
        pl.kernel,
        out_type=jax.ShapeDtypeStruct((NW, L), jnp.float32),
        mesh=mesh,
        compiler_params=pltpu.CompilerParams(
            needs_layout_passes=False, use_tc_tiling_on_sc=False),
        scratch_types=[
            pltpu.VMEM((MAXIT, 3 * K), jnp.int32),
            pltpu.VMEM((2, 3 * K, ROWD), jnp.float32),
            pltpu.VMEM((B,), jnp.float32),
            pltpu.VMEM((NB, L), jnp.float32),
            pltpu.VMEM((L,), jnp.float32),
            pltpu.SemaphoreType.DMA,
            pltpu.SemaphoreType.DMA,
        ],
    )(_edge_body)
    out = run(table, idxs, maskf)
    return jnp.sum(out)


def _pack(x):
    xh = x.astype(jnp.bfloat16)
    u = lax.bitcast_convert_type(xh, jnp.uint16).astype(jnp.uint32)
    words = u[:B // 2] | (u[B // 2:] << 16)
    return (lax.bitcast_convert_type(words, jnp.float32)
            .transpose(1, 2, 0).reshape(N_VERTS, 3 * B // 2))


def kernel(pred_verts, gt_verts, flag, faces):
    table = jnp.concatenate([_pack(pred_verts), _pack(gt_verts)], axis=1)
    f = faces.astype(jnp.int32)
    pad = NS * (IT0 + IT1) * K - N_FACES
    fp = jnp.concatenate([f, jnp.zeros((pad, 3), jnp.int32)], axis=0)
    n0 = NS * IT0 * K
    f0 = fp[:n0].reshape(NS, IT0, K, 3)
    f1 = jnp.pad(fp[n0:].reshape(NS, IT1, K, 3),
                 ((0, 0), (0, MAXIT - IT1), (0, 0), (0, 0)))
    idxs = (jnp.stack([f0, f1], axis=1)
            .reshape(NW, MAXIT, K, 3)
            .transpose(0, 1, 3, 2)
            .reshape(NW, MAXIT, 3 * K))
    maskf = (flag == 1).astype(jnp.float32)
    maskp = maskf.reshape(2, NG, L).transpose(1, 0, 2).reshape(B)
    return _edge_loss(table, idxs, maskp)
